# baseline (speedup 1.0000x reference)
import functools

import jax
import jax.numpy as jnp
from jax import lax
from jax.experimental import pallas as pl
from jax.experimental.pallas import tpu as pltpu
from jax.experimental.pallas import tpu_sc as plsc

N_NODES_ = 29
E_RAW = 232
E_PAD = 240
N_CHUNKS = E_PAD // 16


def _rsqrt_newton(x):
    i = plsc.bitcast(x, jnp.int32)
    i = jnp.int32(0x5F3759DF) - lax.shift_right_arithmetic(i, jnp.int32(1))
    y = plsc.bitcast(i, jnp.float32)
    for _ in range(4):
        y = y * (1.5 - 0.5 * x * y * y)
    return y


def _sc_build_adj(src_pad, dst_pad):
    mesh = plsc.VectorSubcoreMesh(core_axis_name="c", subcore_axis_name="s")

    @functools.partial(
        pl.kernel,
        mesh=mesh,
        compiler_params=pltpu.CompilerParams(needs_layout_passes=False),
        out_type=jax.ShapeDtypeStruct((32, 32), jnp.float32),
        scratch_types=[
            pltpu.VMEM((E_PAD,), jnp.int32),
            pltpu.VMEM((E_PAD,), jnp.int32),
            pltpu.VMEM((32,), jnp.float32),
            pltpu.VMEM((32,), jnp.float32),
            pltpu.VMEM((32, 32), jnp.float32),
        ],
    )
    def k(src_hbm, dst_hbm, ah_hbm, sv, dv, deg, dinv, ahv):
        cid = lax.axis_index("c")
        sid = lax.axis_index("s")

        @pl.when((cid == 0) & (sid == 0))
        def _():
            pltpu.sync_copy(src_hbm, sv)
            pltpu.sync_copy(dst_hbm, dv)

            lane = lax.iota(jnp.int32, 16)
            zeros = jnp.zeros((16,), jnp.float32)
            ones = jnp.ones((16,), jnp.float32)

            deg[pl.ds(0, 16)] = zeros
            deg[pl.ds(16, 16)] = zeros

            def zero_ah(r, carry):
                ahv[r, pl.ds(0, 16)] = zeros
                ahv[r, pl.ds(16, 16)] = zeros
                return carry

            lax.fori_loop(0, 32, zero_ah, 0)

            def deg_body(c, carry):
                dvec = dv[pl.ds(c * 16, 16)]
                valid = (c * 16 + lane) < E_RAW
                for j in range(16):
                    plsc.addupdate_scatter(
                        deg, [dvec], ones, mask=valid & (lane == j))
                return carry

            lax.fori_loop(0, N_CHUNKS, deg_body, 0)

            deg[pl.ds(0, 16)] = deg[pl.ds(0, 16)] + 1.0
            tail = jnp.where(lane < (N_NODES_ - 16), 1.0, 0.0)
            deg[pl.ds(16, 16)] = deg[pl.ds(16, 16)] + tail

            dinv[pl.ds(0, 16)] = _rsqrt_newton(deg[pl.ds(0, 16)])
            dinv[pl.ds(16, 16)] = _rsqrt_newton(
                jnp.maximum(deg[pl.ds(16, 16)], 1.0))

            def edge_body(c, carry):
                svec = sv[pl.ds(c * 16, 16)]
                dvec = dv[pl.ds(c * 16, 16)]
                nrm = (plsc.load_gather(dinv, [svec]) *
                       plsc.load_gather(dinv, [dvec]))
                valid = (c * 16 + lane) < E_RAW
                for j in range(16):
                    plsc.addupdate_scatter(
                        ahv, [dvec, svec], nrm, mask=valid & (lane == j))
                return carry

            lax.fori_loop(0, N_CHUNKS, edge_body, 0)

            for c in range(2):
                ids = lane + c * 16
                dvv = dinv[pl.ds(c * 16, 16)]
                plsc.addupdate_scatter(
                    ahv, [ids, ids], dvv * dvv, mask=ids < N_NODES_)

            pltpu.sync_copy(ahv, ah_hbm)

    return k(src_pad, dst_pad)


def _tc_body(ft_ref, ah_ref, w1_ref, b1_ref, w2_ref, b2_ref, wf1t_ref,
             bf1_ref, wf2t_ref, bf2_ref, wft_ref, bf_ref, outt_ref):
    dot = functools.partial(jnp.dot, preferred_element_type=jnp.float32)
    relu = lambda v: jnp.maximum(v, 0.0)
    ft = ft_ref[...]
    ah = ah_ref[...]
    ah29 = ah[:, :N_NODES_]
    f0 = ft[0]
    f1 = ft[1]
    f2 = ft[2]
    s0 = f0 * w1_ref[0, 0] + f1 * w1_ref[1, 0] + f2 * w1_ref[2, 0]
    s1 = f0 * w1_ref[0, 1] + f1 * w1_ref[1, 1] + f2 * w1_ref[2, 1]
    h10 = relu(dot(ah29, s0) + b1_ref[0])
    h11 = relu(dot(ah29, s1) + b1_ref[1])
    t = h10 * w2_ref[0, 0] + h11 * w2_ref[1, 0]
    h2 = relu(dot(ah, t) + b2_ref[0])[:N_NODES_]
    h3 = relu(dot(wf1t_ref[...], h2) + bf1_ref[...])
    h4 = relu(dot(wf2t_ref[...], h3) + bf2_ref[...])
    outt_ref[...] = dot(wft_ref[...], h4) + bf_ref[...]


def _dense_chain_t(ft, ah, w1, b1, w2, b2, wf1t, bf1c, wf2t, bf2c, wft, bfc,
                   block_b):
    b_total = ft.shape[2]
    grid = (b_total // block_b,)
    vfull = lambda shape: pl.BlockSpec(shape, lambda i: tuple(0 for _ in shape))
    sfull = lambda shape: pl.BlockSpec(
        shape, lambda i: tuple(0 for _ in shape), memory_space=pltpu.SMEM)
    return pl.pallas_call(
        _tc_body,
        grid=grid,
        in_specs=[
            pl.BlockSpec((3, N_NODES_, block_b), lambda i: (0, 0, i)),
            vfull((32, 32)),
            sfull((3, 2)),
            sfull((2,)),
            sfull((2, 1)),
            sfull((1,)),
            vfull((128, 29)),
            vfull((128, 1)),
            vfull((128, 128)),
            vfull((128, 1)),
            vfull((1296, 128)),
            vfull((1296, 1)),
        ],
        out_specs=pl.BlockSpec((1296, block_b), lambda i: (0, i)),
        out_shape=jax.ShapeDtypeStruct((1296, b_total), jnp.float32),
        compiler_params=pltpu.CompilerParams(
            dimension_semantics=("arbitrary",)),
    )(ft, ah, w1, b1, w2, b2, wf1t, bf1c, wf2t, bf2c, wft, bfc)


def kernel(feature, edge_index, W1, b1, W2, b2, Wf1, bf1, Wf2, bf2, Wf, bf):
    src_pad = jnp.pad(edge_index[0], (0, E_PAD - E_RAW)).astype(jnp.int32)
    dst_pad = jnp.pad(edge_index[1], (0, E_PAD - E_RAW)).astype(jnp.int32)
    ah = _sc_build_adj(src_pad, dst_pad)

    ft = jnp.transpose(feature, (2, 1, 0))
    outt = _dense_chain_t(
        ft, ah, W1, b1, W2, b2,
        Wf1.T, bf1.reshape(128, 1), Wf2.T, bf2.reshape(128, 1),
        Wf.T, bf.reshape(1296, 1), block_b=4096)
    return outt.T


# speedup vs baseline: 1.1299x; 1.1299x over previous
import functools

import jax
import jax.numpy as jnp
from jax import lax
from jax.experimental import pallas as pl
from jax.experimental.pallas import tpu as pltpu
from jax.experimental.pallas import tpu_sc as plsc

N_NODES_ = 29
E_RAW = 232
E_PAD = 240
N_CHUNKS = E_PAD // 16


def _rsqrt_newton(x):
    i = plsc.bitcast(x, jnp.int32)
    i = jnp.int32(0x5F3759DF) - lax.shift_right_arithmetic(i, jnp.int32(1))
    y = plsc.bitcast(i, jnp.float32)
    for _ in range(4):
        y = y * (1.5 - 0.5 * x * y * y)
    return y


def _sc_build_adj(src_pad, dst_pad):
    mesh = plsc.VectorSubcoreMesh(core_axis_name="c", subcore_axis_name="s")

    @functools.partial(
        pl.kernel,
        mesh=mesh,
        compiler_params=pltpu.CompilerParams(needs_layout_passes=False),
        out_type=jax.ShapeDtypeStruct((32, 32), jnp.float32),
        scratch_types=[
            pltpu.VMEM((E_PAD,), jnp.int32),
            pltpu.VMEM((E_PAD,), jnp.int32),
            pltpu.VMEM((32,), jnp.float32),
            pltpu.VMEM((32,), jnp.float32),
            pltpu.VMEM((32, 32), jnp.float32),
        ],
    )
    def k(src_hbm, dst_hbm, ah_hbm, sv, dv, deg, dinv, ahv):
        cid = lax.axis_index("c")
        sid = lax.axis_index("s")

        @pl.when((cid == 0) & (sid == 0))
        def _():
            pltpu.sync_copy(src_hbm, sv)
            pltpu.sync_copy(dst_hbm, dv)

            lane = lax.iota(jnp.int32, 16)
            zeros = jnp.zeros((16,), jnp.float32)
            ones = jnp.ones((16,), jnp.float32)

            deg[pl.ds(0, 16)] = zeros
            deg[pl.ds(16, 16)] = zeros

            def zero_ah(r, carry):
                ahv[r, pl.ds(0, 16)] = zeros
                ahv[r, pl.ds(16, 16)] = zeros
                return carry

            lax.fori_loop(0, 32, zero_ah, 0)

            def deg_body(c, carry):
                dvec = dv[pl.ds(c * 16, 16)]
                valid = (c * 16 + lane) < E_RAW
                for j in range(16):
                    plsc.addupdate_scatter(
                        deg, [dvec], ones, mask=valid & (lane == j))
                return carry

            lax.fori_loop(0, N_CHUNKS, deg_body, 0)

            deg[pl.ds(0, 16)] = deg[pl.ds(0, 16)] + 1.0
            tail = jnp.where(lane < (N_NODES_ - 16), 1.0, 0.0)
            deg[pl.ds(16, 16)] = deg[pl.ds(16, 16)] + tail

            dinv[pl.ds(0, 16)] = _rsqrt_newton(deg[pl.ds(0, 16)])
            dinv[pl.ds(16, 16)] = _rsqrt_newton(
                jnp.maximum(deg[pl.ds(16, 16)], 1.0))

            def edge_body(c, carry):
                svec = sv[pl.ds(c * 16, 16)]
                dvec = dv[pl.ds(c * 16, 16)]
                nrm = (plsc.load_gather(dinv, [svec]) *
                       plsc.load_gather(dinv, [dvec]))
                valid = (c * 16 + lane) < E_RAW
                for j in range(16):
                    plsc.addupdate_scatter(
                        ahv, [dvec, svec], nrm, mask=valid & (lane == j))
                return carry

            lax.fori_loop(0, N_CHUNKS, edge_body, 0)

            for c in range(2):
                ids = lane + c * 16
                dvv = dinv[pl.ds(c * 16, 16)]
                plsc.addupdate_scatter(
                    ahv, [ids, ids], dvv * dvv, mask=ids < N_NODES_)

            pltpu.sync_copy(ahv, ah_hbm)

    return k(src_pad, dst_pad)


def _tc_body(ft_ref, ah_ref, w1_ref, b1_ref, w2_ref, b2_ref, wf1t_ref,
             bf1_ref, wf2t_ref, bf2_ref, wft_ref, bf_ref, outt_ref):
    dot = functools.partial(jnp.dot, preferred_element_type=jnp.float32)
    relu = lambda v: jnp.maximum(v, 0.0)
    ft = ft_ref[...]
    ah = ah_ref[...]
    ah29 = ah[:, :N_NODES_]
    f0 = ft[0]
    f1 = ft[1]
    f2 = ft[2]
    s0 = f0 * w1_ref[0, 0] + f1 * w1_ref[1, 0] + f2 * w1_ref[2, 0]
    s1 = f0 * w1_ref[0, 1] + f1 * w1_ref[1, 1] + f2 * w1_ref[2, 1]
    h10 = relu(dot(ah29, s0) + b1_ref[0])
    h11 = relu(dot(ah29, s1) + b1_ref[1])
    t = h10 * w2_ref[0, 0] + h11 * w2_ref[1, 0]
    h2 = relu(dot(ah, t) + b2_ref[0])[:N_NODES_]
    h3 = relu(dot(wf1t_ref[...], h2) + bf1_ref[...][:, None])
    h4 = relu(dot(wf2t_ref[...], h3) + bf2_ref[...][:, None])
    outt_ref[...] = dot(wft_ref[...], h4) + bf_ref[...][:, None]


def _dense_chain_t(ft, ah, w1, b1, w2, b2, wf1t, bf1c, wf2t, bf2c, wft, bfc,
                   block_b):
    b_total = ft.shape[2]
    grid = (b_total // block_b,)
    vfull = lambda shape: pl.BlockSpec(shape, lambda i: tuple(0 for _ in shape))
    sfull = lambda shape: pl.BlockSpec(
        shape, lambda i: tuple(0 for _ in shape), memory_space=pltpu.SMEM)
    return pl.pallas_call(
        _tc_body,
        grid=grid,
        in_specs=[
            pl.BlockSpec((3, N_NODES_, block_b), lambda i: (0, 0, i)),
            vfull((32, 32)),
            sfull((3, 2)),
            sfull((2,)),
            sfull((2, 1)),
            sfull((1,)),
            vfull((128, 29)),
            vfull((128,)),
            vfull((128, 128)),
            vfull((128,)),
            vfull((1296, 128)),
            vfull((1296,)),
        ],
        out_specs=pl.BlockSpec((1296, block_b), lambda i: (0, i)),
        out_shape=jax.ShapeDtypeStruct((1296, b_total), jnp.float32),
        compiler_params=pltpu.CompilerParams(
            dimension_semantics=("arbitrary",)),
    )(ft, ah, w1, b1, w2, b2, wf1t, bf1c, wf2t, bf2c, wft, bfc)


def kernel(feature, edge_index, W1, b1, W2, b2, Wf1, bf1, Wf2, bf2, Wf, bf):
    src_pad = jnp.pad(edge_index[0], (0, E_PAD - E_RAW)).astype(jnp.int32)
    dst_pad = jnp.pad(edge_index[1], (0, E_PAD - E_RAW)).astype(jnp.int32)
    ah = _sc_build_adj(src_pad, dst_pad)

    ft = jnp.transpose(feature, (2, 1, 0))
    outt = _dense_chain_t(
        ft, ah, W1, b1, W2, b2,
        Wf1.T, bf1, Wf2.T, bf2,
        Wf.T, bf, block_b=2048)
    return outt.T
